# Initial kernel scaffold; baseline (speedup 1.0000x reference)
import jax
import jax.numpy as jnp
from jax.experimental import pallas as pl


def kernel(node_graph_feat, neighbors, node, W1, a_src1, a_dst1, b1, W2, a_src2, a_dst2, b2, Wc, bc):
    raise NotImplementedError("write your pallas kernel here")


# speedup vs baseline: 17.7443x; 17.7443x over previous
import functools

import jax
import jax.numpy as jnp
from jax import lax
from jax.experimental import pallas as pl
from jax.experimental.pallas import tpu as pltpu, tpu_sc as plsc

N = 10000
D = 128
C = 32
B = 4096

NPAD = 10240
NC = 2
NS = 16
NW = NC * NS
G = 64
RPT = NPAD // NS
DENW = 16
MRPT = NPAD // NW


def _mm_body(x_ref, w_ref, o_ref):
    o_ref[...] = jnp.dot(x_ref[...], w_ref[...],
                         preferred_element_type=jnp.float32)


def _tc_proj(x, wext):
    mblk = NPAD // 4
    return pl.pallas_call(
        _mm_body,
        grid=(4,),
        in_specs=[
            pl.BlockSpec((mblk, D), lambda i: (i, 0)),
            pl.BlockSpec((D, 2 * D), lambda i: (0, 0)),
        ],
        out_specs=pl.BlockSpec((mblk, 2 * D), lambda i: (i, 0)),
        out_shape=jax.ShapeDtypeStruct((NPAD, 2 * D), jnp.float32),
    )(x, wext)


def _tc_norm_mm(num, den, b, w, bias_out=None):
    mblk = NPAD // 4
    kn = w.shape[1]
    if bias_out is None:
        def body(n_ref, d_ref, b_ref, w_ref, o_ref):
            x = ((n_ref[0] + n_ref[1]) / (d_ref[:, 0:1] + 1e-16)
                 + b_ref[...])
            o_ref[...] = jnp.dot(x, w_ref[...],
                                 preferred_element_type=jnp.float32)
        extra = ()
    else:
        def body(n_ref, d_ref, b_ref, w_ref, bo_ref, o_ref):
            x = ((n_ref[0] + n_ref[1]) / (d_ref[:, 0:1] + 1e-16)
                 + b_ref[...])
            o_ref[...] = jnp.dot(
                x, w_ref[...], preferred_element_type=jnp.float32
            ) + bo_ref[...]
        extra = (pl.BlockSpec((kn,), lambda i: (0,)),)
    args = (num, den, b, w) + (() if bias_out is None else (bias_out,))
    return pl.pallas_call(
        body,
        grid=(4,),
        in_specs=[
            pl.BlockSpec((NC, mblk, D), lambda i: (0, i, 0)),
            pl.BlockSpec((mblk, DENW), lambda i: (i, 0)),
            pl.BlockSpec((D,), lambda i: (0,)),
            pl.BlockSpec((D, kn), lambda i: (0, 0)),
        ] + list(extra),
        out_specs=pl.BlockSpec((mblk, kn), lambda i: (i, 0)),
        out_shape=jax.ShapeDtypeStruct((NPAD, kn), jnp.float32),
    )(*args)


def _sc_mesh():
    return plsc.VectorSubcoreMesh(core_axis_name="c", subcore_axis_name="s")


_SC_PARAMS = pltpu.CompilerParams(needs_layout_passes=False)


def _edge_pass(e_pad):
    chunk = e_pad // NW
    steps = chunk // G

    @functools.partial(
        pl.kernel,
        out_type=(
            jax.ShapeDtypeStruct((NC, NPAD, D), jnp.float32),
            jax.ShapeDtypeStruct((NW * NPAD,), jnp.float32),
        ),
        mesh=_sc_mesh(),
        scratch_types=[
            pltpu.VMEM((NPAD,), jnp.float32),
            pltpu.VMEM((NPAD,), jnp.float32),
            pltpu.VMEM((NPAD,), jnp.float32),
            pltpu.VMEM((G,), jnp.int32),
            pltpu.VMEM((G,), jnp.int32),
            pltpu.VMEM((G, D), jnp.float32),
            pltpu.VMEM((G, D), jnp.float32),
            pltpu.VMEM((G + 16,), jnp.float32),
            pltpu.VMEM_SHARED((NPAD, D), jnp.float32),
            pltpu.SemaphoreType.DMA,
        ],
        compiler_params=_SC_PARAMS,
    )
    def k(h_hbm, as_hbm, ad_hbm, src_hbm, dst_hbm, num_hbm, dparts_hbm,
          as_v, ad_v, den_v, sidx, didx, rows_raw, rows_out, exb, acc, sem):
        cid = lax.axis_index("c")
        sid = lax.axis_index("s")
        wid = sid * NC + cid
        zeros16 = jnp.zeros((16,), jnp.float32)

        def zrow(i, _):
            for c in range(D // 16):
                rows_out[i, pl.ds(c * 16, 16)] = zeros16
            return 0
        lax.fori_loop(0, G, zrow, 0)
        for off in range(0, RPT, G):
            pltpu.sync_copy(rows_out, acc.at[pl.ds(sid * RPT + off, G)])

        def zden(i, _):
            den_v[pl.ds(i * 16, 16)] = zeros16
            return 0
        lax.fori_loop(0, NPAD // 16, zden, 0)
        plsc.subcore_barrier()

        pltpu.sync_copy(as_hbm, as_v)
        pltpu.sync_copy(ad_hbm, ad_v)

        def step(t, _):
            base = wid * chunk + t * G
            pltpu.sync_copy(src_hbm.at[pl.ds(base, G)], sidx)
            pltpu.sync_copy(dst_hbm.at[pl.ds(base, G)], didx)
            pltpu.async_copy(h_hbm.at[sidx], rows_raw, sem).wait()

            def grp(g, _):
                s16 = sidx[pl.ds(g * 16, 16)]
                d16 = didx[pl.ds(g * 16, 16)]
                e = (plsc.load_gather(as_v, [s16])
                     + plsc.load_gather(ad_v, [d16]))
                e = jnp.where(e >= 0.0, e, 0.2 * e)
                ex = jnp.exp(e)
                exb[pl.ds(g * 16, 16)] = ex
                plsc.addupdate_scatter(den_v, [d16], ex)
                return 0
            lax.fori_loop(0, G // 16, grp, 0)

            def row(i, _):
                exi = exb[pl.ds(i, 16)][0]
                for c in range(D // 16):
                    rows_out[i, pl.ds(c * 16, 16)] = (
                        rows_raw[i, pl.ds(c * 16, 16)] * exi)
                return 0
            lax.fori_loop(0, G, row, 0)

            pltpu.sync_copy(rows_out, acc.at[didx], add=True)
            return 0
        lax.fori_loop(0, steps, step, 0)

        pltpu.sync_copy(den_v, dparts_hbm.at[pl.ds(wid * NPAD, NPAD)])
        plsc.subcore_barrier()
        for off in range(0, RPT, G):
            rr = sid * RPT + off
            pltpu.sync_copy(acc.at[pl.ds(rr, G)], rows_out)
            pltpu.sync_copy(rows_out, num_hbm.at[cid, pl.ds(rr, G)])

    return k


@functools.partial(
    pl.kernel,
    out_type=jax.ShapeDtypeStruct((NPAD * DENW,), jnp.float32),
    mesh=_sc_mesh(),
    scratch_types=[
        pltpu.VMEM((NW * MRPT,), jnp.float32),
        pltpu.VMEM((MRPT + 16,), jnp.float32),
        pltpu.VMEM((MRPT * DENW,), jnp.float32),
    ],
    compiler_params=_SC_PARAMS,
)
def _den_merge(dparts_hbm, den_hbm, buf, dm, dbuf):
    wid = lax.axis_index("s") * NC + lax.axis_index("c")
    rb = wid * MRPT
    for i in range(NW):
        pltpu.sync_copy(dparts_hbm.at[pl.ds(i * NPAD + rb, MRPT)],
                        buf.at[pl.ds(i * MRPT, MRPT)])

    def csum(c, _):
        tot = buf[pl.ds(c * 16, 16)]
        for i in range(1, NW):
            tot = tot + buf[pl.ds(i * MRPT + c * 16, 16)]
        dm[pl.ds(c * 16, 16)] = tot
        return 0
    lax.fori_loop(0, MRPT // 16, csum, 0)

    lane0 = lax.iota(jnp.int32, 16) == 0

    def dexp(j, _):
        dj = dm[pl.ds(j, 16)][0]
        dbuf[pl.ds(j * DENW, 16)] = jnp.where(lane0, dj, 0.0)
        return 0
    lax.fori_loop(0, MRPT, dexp, 0)
    pltpu.sync_copy(dbuf, den_hbm.at[pl.ds(rb * DENW, MRPT * DENW)])


def _sc_row_gather(width):
    rows = B // NW

    @functools.partial(
        pl.kernel,
        out_type=jax.ShapeDtypeStruct((B, width), jnp.float32),
        mesh=_sc_mesh(),
        scratch_types=[
            pltpu.VMEM((rows,), jnp.int32),
            pltpu.VMEM((rows, width), jnp.float32),
            pltpu.SemaphoreType.DMA,
        ],
        compiler_params=_SC_PARAMS,
    )
    def k(tab_hbm, node_hbm, out_hbm, nidx, rows_v, sem):
        cid = lax.axis_index("c")
        sid = lax.axis_index("s")
        base = (sid * NC + cid) * rows
        pltpu.sync_copy(node_hbm.at[pl.ds(base, rows)], nidx)
        pltpu.async_copy(tab_hbm.at[nidx], rows_v, sem).wait()
        pltpu.sync_copy(rows_v, out_hbm.at[pl.ds(base, rows)])

    return k


def _wext(w, a_s, a_d):
    cols = [w.T, (w.T @ a_s)[:, None], (w.T @ a_d)[:, None],
            jnp.zeros((D, D - 2), jnp.float32)]
    return jnp.concatenate(cols, axis=1)


def kernel(node_graph_feat, neighbors, node, W1, a_src1, a_dst1, b1,
           W2, a_src2, a_dst2, b2, Wc, bc):
    e_tot = neighbors.shape[1] + N
    e_pad = ((e_tot + NW * G - 1) // (NW * G)) * (NW * G)
    loop = jnp.arange(N, dtype=jnp.int32)
    src = jnp.concatenate(
        [neighbors[0], loop, jnp.zeros((e_pad - e_tot,), jnp.int32)])
    dst = jnp.concatenate(
        [neighbors[1], loop, jnp.full((e_pad - e_tot,), N, jnp.int32)])

    xp = jnp.pad(node_graph_feat, ((0, NPAD - N), (0, 0)))

    edge_pass = _edge_pass(e_pad)

    hext1 = _tc_proj(xp, _wext(W1, a_src1, a_dst1))
    num1, dp1 = edge_pass(hext1[:, :D], hext1[:, D], hext1[:, D + 1],
                          src, dst)
    den1 = _den_merge(dp1).reshape(NPAD, DENW)

    hext2 = _tc_norm_mm(num1, den1, b1, _wext(W2, a_src2, a_dst2))
    num2, dp2 = edge_pass(hext2[:, :D], hext2[:, D], hext2[:, D + 1],
                          src, dst)
    den2 = _den_merge(dp2).reshape(NPAD, DENW)

    wc_pad = jnp.concatenate([Wc.T, jnp.zeros((D, D - C), jnp.float32)], 1)
    bc_pad = jnp.concatenate([bc, jnp.zeros((D - C,), jnp.float32)])
    pred_all = _tc_norm_mm(num2, den2, b2, wc_pad, bias_out=bc_pad)
    pred = _sc_row_gather(D)(pred_all, node)
    return pred[:, :C]


# speedup vs baseline: 34.8525x; 1.9641x over previous
import functools

import jax
import jax.numpy as jnp
from jax import lax
from jax.experimental import pallas as pl
from jax.experimental.pallas import tpu as pltpu, tpu_sc as plsc

N = 10000
D = 128
C = 32
B = 4096

NPAD = 10240
NC = 2
NS = 16
NW = NC * NS
G = 32
NB = 4
NI = 8
RPT = NPAD // NS
DENW = 16
MRPT = NPAD // NW
ALEN = 10016


def _mm_body(x_ref, w_ref, o_ref):
    o_ref[...] = jnp.dot(x_ref[...], w_ref[...],
                         preferred_element_type=jnp.float32)


def _tc_proj(x, wext):
    mblk = NPAD // 4
    return pl.pallas_call(
        _mm_body,
        grid=(4,),
        in_specs=[
            pl.BlockSpec((mblk, D), lambda i: (i, 0)),
            pl.BlockSpec((D, 2 * D), lambda i: (0, 0)),
        ],
        out_specs=pl.BlockSpec((mblk, 2 * D), lambda i: (i, 0)),
        out_shape=jax.ShapeDtypeStruct((NPAD, 2 * D), jnp.float32),
    )(x, wext)


def _tc_norm_mm(num, den, b, w, bias_out=None):
    mblk = NPAD // 4
    kn = w.shape[1]
    if bias_out is None:
        def body(n_ref, d_ref, b_ref, w_ref, o_ref):
            x = ((n_ref[0] + n_ref[1]) / (d_ref[:, 0:1] + 1e-16)
                 + b_ref[...])
            o_ref[...] = jnp.dot(x, w_ref[...],
                                 preferred_element_type=jnp.float32)
        extra = ()
    else:
        def body(n_ref, d_ref, b_ref, w_ref, bo_ref, o_ref):
            x = ((n_ref[0] + n_ref[1]) / (d_ref[:, 0:1] + 1e-16)
                 + b_ref[...])
            o_ref[...] = jnp.dot(
                x, w_ref[...], preferred_element_type=jnp.float32
            ) + bo_ref[...]
        extra = (pl.BlockSpec((kn,), lambda i: (0,)),)
    args = (num, den, b, w) + (() if bias_out is None else (bias_out,))
    return pl.pallas_call(
        body,
        grid=(4,),
        in_specs=[
            pl.BlockSpec((NC, mblk, D), lambda i: (0, i, 0)),
            pl.BlockSpec((mblk, DENW), lambda i: (i, 0)),
            pl.BlockSpec((D,), lambda i: (0,)),
            pl.BlockSpec((D, kn), lambda i: (0, 0)),
        ] + list(extra),
        out_specs=pl.BlockSpec((mblk, kn), lambda i: (i, 0)),
        out_shape=jax.ShapeDtypeStruct((NPAD, kn), jnp.float32),
    )(*args)


def _sc_mesh():
    return plsc.VectorSubcoreMesh(core_axis_name="c", subcore_axis_name="s")


_SC_PARAMS = pltpu.CompilerParams(needs_layout_passes=False)


def _edge_pass(e_pad):
    chunk = e_pad // NW
    steps = chunk // G

    @functools.partial(
        pl.kernel,
        out_type=(
            jax.ShapeDtypeStruct((NC, NPAD, D), jnp.float32),
            jax.ShapeDtypeStruct((NW * NPAD,), jnp.float32),
        ),
        mesh=_sc_mesh(),
        scratch_types=[
            pltpu.VMEM((ALEN,), jnp.float32),
            pltpu.VMEM((ALEN,), jnp.float32),
            pltpu.VMEM((NPAD,), jnp.float32),
            pltpu.VMEM((G + 16,), jnp.float32),
        ]
        + [pltpu.VMEM((G, D), jnp.float32)] * NB
        + [pltpu.VMEM((G,), jnp.int32)] * NI
        + [pltpu.VMEM((G,), jnp.int32)] * NI
        + [pltpu.VMEM_SHARED((NPAD, D), jnp.float32)]
        + [pltpu.SemaphoreType.DMA] * (2 * NB + NI),
        compiler_params=_SC_PARAMS,
    )
    def k(h_hbm, as_hbm, ad_hbm, src_hbm, dst_hbm, num_hbm, dparts_hbm,
          as_v, ad_v, den_v, exb, *ring):
        rows = ring[:NB]
        sidx = ring[NB:NB + NI]
        didx = ring[NB + NI:NB + 2 * NI]
        acc = ring[NB + 2 * NI]
        semg = ring[NB + 2 * NI + 1:NB + 2 * NI + 1 + NB]
        sems = ring[NB + 2 * NI + 1 + NB:NB + 2 * NI + 1 + 2 * NB]
        semi = ring[NB + 2 * NI + 1 + 2 * NB:]
        cid = lax.axis_index("c")
        sid = lax.axis_index("s")
        wid = sid * NC + cid
        cbase = wid * chunk
        zeros16 = jnp.zeros((16,), jnp.float32)

        def zrow(i, _):
            for c in range(D // 16):
                rows[0][i, pl.ds(c * 16, 16)] = zeros16
            return 0
        lax.fori_loop(0, G, zrow, 0)
        for off in range(0, RPT, G):
            pltpu.sync_copy(rows[0], acc.at[pl.ds(sid * RPT + off, G)])

        def zden(i, _):
            den_v[pl.ds(i * 16, 16)] = zeros16
            return 0
        lax.fori_loop(0, NPAD // 16, zden, 0)
        plsc.subcore_barrier()

        pltpu.sync_copy(as_hbm, as_v)
        pltpu.sync_copy(ad_hbm, ad_v)

        def idx_issue(s, j):
            base = cbase + s * G
            pltpu.async_copy(src_hbm.at[pl.ds(base, G)], sidx[j], semi[j])
            pltpu.async_copy(dst_hbm.at[pl.ds(base, G)], didx[j], semi[j])

        def idx_wait(j):
            pltpu.make_async_copy(
                src_hbm.at[pl.ds(0, G)], sidx[j], semi[j]).wait()
            pltpu.make_async_copy(
                dst_hbm.at[pl.ds(0, G)], didx[j], semi[j]).wait()

        def compute(b, j):
            for g in range(G // 16):
                s16 = sidx[j][pl.ds(g * 16, 16)]
                d16 = didx[j][pl.ds(g * 16, 16)]
                e = (plsc.load_gather(as_v, [s16])
                     + plsc.load_gather(ad_v, [d16]))
                e = jnp.where(e >= 0.0, e, 0.2 * e)
                ex = jnp.exp(e)
                exb[pl.ds(g * 16, 16)] = ex
                plsc.addupdate_scatter(den_v, [d16], ex)

            def row(i4, _):
                for v in range(4):
                    i = i4 * 4 + v
                    exi = exb[pl.ds(i, 16)][0]
                    for c in range(D // 16):
                        sl = pl.ds(c * 16, 16)
                        rows[b][i, sl] = rows[b][i, sl] * exi
                return 0
            lax.fori_loop(0, G // 4, row, 0)

        def phase(s, p, w_scat, i_idx, i_gath):
            b, j = p % NB, p % NI
            pltpu.make_async_copy(h_hbm.at[sidx[j]], rows[b],
                                  semg[b]).wait()
            compute(b, j)
            pltpu.async_copy(rows[b], acc.at[didx[j]], sems[b], add=True)
            if w_scat:
                b2, j2 = (p + 2) % NB, (p + 6) % NI
                pltpu.make_async_copy(rows[b2], acc.at[didx[j2]],
                                      sems[b2]).wait()
            if i_idx:
                idx_issue(s + 3, (p + 3) % NI)
            if i_gath:
                b2, j2 = (p + 2) % NB, (p + 2) % NI
                idx_wait(j2)
                pltpu.async_copy(h_hbm.at[sidx[j2]], rows[b2], semg[b2])

        for s0 in range(3):
            idx_issue(s0, s0)
        for s0 in range(2):
            idx_wait(s0)
            pltpu.async_copy(h_hbm.at[sidx[s0]], rows[s0], semg[s0])

        for s0 in range(NI):
            phase(s0, s0, s0 >= 2, s0 + 3 < steps, s0 + 2 < steps)

        nq = steps // NI

        def superstep(q, _):
            for p in range(NI):
                phase(q * NI + p, p, True, True, True)
            return 0
        lax.fori_loop(1, nq, superstep, 0)

        for s0 in range(nq * NI, steps):
            p = s0 % NI
            phase(s0, p, True, s0 + 3 < steps, s0 + 2 < steps)
        for s0 in (steps - 2, steps - 1):
            b, j = s0 % NB, s0 % NI
            pltpu.make_async_copy(rows[b], acc.at[didx[j]],
                                  sems[b]).wait()

        pltpu.sync_copy(den_v, dparts_hbm.at[pl.ds(wid * NPAD, NPAD)])
        plsc.subcore_barrier()
        for off in range(0, RPT, G):
            rr = sid * RPT + off
            pltpu.sync_copy(acc.at[pl.ds(rr, G)], rows[0])
            pltpu.sync_copy(rows[0], num_hbm.at[cid, pl.ds(rr, G)])

    return k


@functools.partial(
    pl.kernel,
    out_type=jax.ShapeDtypeStruct((NPAD * DENW,), jnp.float32),
    mesh=_sc_mesh(),
    scratch_types=[
        pltpu.VMEM((NW * MRPT,), jnp.float32),
        pltpu.VMEM((MRPT + 16,), jnp.float32),
        pltpu.VMEM((MRPT * DENW,), jnp.float32),
    ],
    compiler_params=_SC_PARAMS,
)
def _den_merge(dparts_hbm, den_hbm, buf, dm, dbuf):
    wid = lax.axis_index("s") * NC + lax.axis_index("c")
    rb = wid * MRPT
    for i in range(NW):
        pltpu.sync_copy(dparts_hbm.at[pl.ds(i * NPAD + rb, MRPT)],
                        buf.at[pl.ds(i * MRPT, MRPT)])

    def csum(c, _):
        tot = buf[pl.ds(c * 16, 16)]
        for i in range(1, NW):
            tot = tot + buf[pl.ds(i * MRPT + c * 16, 16)]
        dm[pl.ds(c * 16, 16)] = tot
        return 0
    lax.fori_loop(0, MRPT // 16, csum, 0)

    lane0 = lax.iota(jnp.int32, 16) == 0

    def dexp(j, _):
        dj = dm[pl.ds(j, 16)][0]
        dbuf[pl.ds(j * DENW, 16)] = jnp.where(lane0, dj, 0.0)
        return 0
    lax.fori_loop(0, MRPT, dexp, 0)
    pltpu.sync_copy(dbuf, den_hbm.at[pl.ds(rb * DENW, MRPT * DENW)])


def _sc_row_gather(width):
    rows = B // NW

    @functools.partial(
        pl.kernel,
        out_type=jax.ShapeDtypeStruct((B, width), jnp.float32),
        mesh=_sc_mesh(),
        scratch_types=[
            pltpu.VMEM((rows,), jnp.int32),
            pltpu.VMEM((rows, width), jnp.float32),
            pltpu.SemaphoreType.DMA,
        ],
        compiler_params=_SC_PARAMS,
    )
    def k(tab_hbm, node_hbm, out_hbm, nidx, rows_v, sem):
        cid = lax.axis_index("c")
        sid = lax.axis_index("s")
        base = (sid * NC + cid) * rows
        pltpu.sync_copy(node_hbm.at[pl.ds(base, rows)], nidx)
        pltpu.async_copy(tab_hbm.at[nidx], rows_v, sem).wait()
        pltpu.sync_copy(rows_v, out_hbm.at[pl.ds(base, rows)])

    return k


def _wext(w, a_s, a_d):
    cols = [w.T, (w.T @ a_s)[:, None], (w.T @ a_d)[:, None],
            jnp.zeros((D, D - 2), jnp.float32)]
    return jnp.concatenate(cols, axis=1)


def kernel(node_graph_feat, neighbors, node, W1, a_src1, a_dst1, b1,
           W2, a_src2, a_dst2, b2, Wc, bc):
    e_tot = neighbors.shape[1] + N
    e_pad = ((e_tot + NW * G - 1) // (NW * G)) * (NW * G)
    loop = jnp.arange(N, dtype=jnp.int32)
    src = jnp.concatenate(
        [neighbors[0], loop, jnp.zeros((e_pad - e_tot,), jnp.int32)])
    dst = jnp.concatenate(
        [neighbors[1], loop, jnp.full((e_pad - e_tot,), N, jnp.int32)])

    xp = jnp.pad(node_graph_feat, ((0, NPAD - N), (0, 0)))

    edge_pass = _edge_pass(e_pad)

    hext1 = _tc_proj(xp, _wext(W1, a_src1, a_dst1))
    num1, dp1 = edge_pass(hext1[:, :D], hext1[:ALEN, D],
                          hext1[:ALEN, D + 1], src, dst)
    den1 = _den_merge(dp1).reshape(NPAD, DENW)

    hext2 = _tc_norm_mm(num1, den1, b1, _wext(W2, a_src2, a_dst2))
    num2, dp2 = edge_pass(hext2[:, :D], hext2[:ALEN, D],
                          hext2[:ALEN, D + 1], src, dst)
    den2 = _den_merge(dp2).reshape(NPAD, DENW)

    wc_pad = jnp.concatenate([Wc.T, jnp.zeros((D, D - C), jnp.float32)], 1)
    bc_pad = jnp.concatenate([bc, jnp.zeros((D - C,), jnp.float32)])
    pred_all = _tc_norm_mm(num2, den2, b2, wc_pad, bias_out=bc_pad)
    pred = _sc_row_gather(D)(pred_all, node)
    return pred[:, :C]


# speedup vs baseline: 39.6347x; 1.1372x over previous
import functools

import jax
import jax.numpy as jnp
from jax import lax
from jax.experimental import pallas as pl
from jax.experimental.pallas import tpu as pltpu, tpu_sc as plsc

N = 10000
D = 128
C = 32
B = 4096

NPAD = 10240
NC = 2
NS = 16
NW = NC * NS
G = 32
NB = 4
NI = 8
RPT = NPAD // NS
DENW = 16
MRPT = NPAD // NW
ALEN = 10016


def _mm_body(x_ref, w_ref, o_ref):
    o_ref[...] = jnp.dot(x_ref[...], w_ref[...],
                         preferred_element_type=jnp.float32)


def _tc_proj(x, wext):
    mblk = NPAD // 4
    return pl.pallas_call(
        _mm_body,
        grid=(4,),
        in_specs=[
            pl.BlockSpec((mblk, D), lambda i: (i, 0)),
            pl.BlockSpec((D, 2 * D), lambda i: (0, 0)),
        ],
        out_specs=pl.BlockSpec((mblk, 2 * D), lambda i: (i, 0)),
        out_shape=jax.ShapeDtypeStruct((NPAD, 2 * D), jnp.float32),
    )(x, wext)


def _tc_norm_mm(num, dparts, b, w, bias_out=None):
    mblk = NPAD // 4
    kn = w.shape[1]

    def _x(n_ref, d_ref, b_ref):
        den = lax.dot_general(
            d_ref[...], jnp.ones((NW, 1), jnp.float32),
            (((0,), (0,)), ((), ())), preferred_element_type=jnp.float32)
        return (n_ref[0] + n_ref[1]) / (den + 1e-16) + b_ref[...]

    if bias_out is None:
        def body(n_ref, d_ref, b_ref, w_ref, o_ref):
            o_ref[...] = jnp.dot(_x(n_ref, d_ref, b_ref), w_ref[...],
                                 preferred_element_type=jnp.float32)
        extra = ()
    else:
        def body(n_ref, d_ref, b_ref, w_ref, bo_ref, o_ref):
            o_ref[...] = jnp.dot(
                _x(n_ref, d_ref, b_ref), w_ref[...],
                preferred_element_type=jnp.float32) + bo_ref[...]
        extra = (pl.BlockSpec((kn,), lambda i: (0,)),)
    args = (num, dparts, b, w) + (() if bias_out is None else (bias_out,))
    return pl.pallas_call(
        body,
        grid=(4,),
        in_specs=[
            pl.BlockSpec((NC, mblk, D), lambda i: (0, i, 0)),
            pl.BlockSpec((NW, mblk), lambda i: (0, i)),
            pl.BlockSpec((D,), lambda i: (0,)),
            pl.BlockSpec((D, kn), lambda i: (0, 0)),
        ] + list(extra),
        out_specs=pl.BlockSpec((mblk, kn), lambda i: (i, 0)),
        out_shape=jax.ShapeDtypeStruct((NPAD, kn), jnp.float32),
    )(*args)


def _sc_mesh():
    return plsc.VectorSubcoreMesh(core_axis_name="c", subcore_axis_name="s")


_SC_PARAMS = pltpu.CompilerParams(needs_layout_passes=False)


def _edge_pass(e_pad):
    chunk = e_pad // NW
    steps = chunk // G

    @functools.partial(
        pl.kernel,
        out_type=(
            jax.ShapeDtypeStruct((NC, NPAD, D), jnp.float32),
            jax.ShapeDtypeStruct((NW * NPAD,), jnp.float32),
        ),
        mesh=_sc_mesh(),
        scratch_types=[
            pltpu.VMEM((ALEN,), jnp.float32),
            pltpu.VMEM((ALEN,), jnp.float32),
            pltpu.VMEM((NPAD,), jnp.float32),
            pltpu.VMEM((G + 16,), jnp.float32),
        ]
        + [pltpu.VMEM((G, D), jnp.float32)] * NB
        + [pltpu.VMEM((G,), jnp.int32)] * NI
        + [pltpu.VMEM((G,), jnp.int32)] * NI
        + [pltpu.VMEM_SHARED((NPAD, D), jnp.float32)]
        + [pltpu.SemaphoreType.DMA] * (2 * NB + NI),
        compiler_params=_SC_PARAMS,
    )
    def k(h_hbm, as_hbm, ad_hbm, src_hbm, dst_hbm, num_hbm, dparts_hbm,
          as_v, ad_v, den_v, exb, *ring):
        rows = ring[:NB]
        sidx = ring[NB:NB + NI]
        didx = ring[NB + NI:NB + 2 * NI]
        acc = ring[NB + 2 * NI]
        semg = ring[NB + 2 * NI + 1:NB + 2 * NI + 1 + NB]
        sems = ring[NB + 2 * NI + 1 + NB:NB + 2 * NI + 1 + 2 * NB]
        semi = ring[NB + 2 * NI + 1 + 2 * NB:]
        cid = lax.axis_index("c")
        sid = lax.axis_index("s")
        wid = sid * NC + cid
        cbase = wid * chunk
        zeros16 = jnp.zeros((16,), jnp.float32)

        def zrow(i, _):
            for c in range(D // 16):
                rows[0][i, pl.ds(c * 16, 16)] = zeros16
            return 0
        lax.fori_loop(0, G, zrow, 0)
        for off in range(0, RPT, G):
            pltpu.sync_copy(rows[0], acc.at[pl.ds(sid * RPT + off, G)])

        def zden(i, _):
            den_v[pl.ds(i * 16, 16)] = zeros16
            return 0
        lax.fori_loop(0, NPAD // 16, zden, 0)
        plsc.subcore_barrier()

        pltpu.sync_copy(as_hbm, as_v)
        pltpu.sync_copy(ad_hbm, ad_v)

        def idx_issue(s, j):
            base = cbase + s * G
            pltpu.async_copy(src_hbm.at[pl.ds(base, G)], sidx[j], semi[j])
            pltpu.async_copy(dst_hbm.at[pl.ds(base, G)], didx[j], semi[j])

        def idx_wait(j):
            pltpu.make_async_copy(
                src_hbm.at[pl.ds(0, G)], sidx[j], semi[j]).wait()
            pltpu.make_async_copy(
                dst_hbm.at[pl.ds(0, G)], didx[j], semi[j]).wait()

        def compute(b, j):
            for g in range(G // 16):
                s16 = sidx[j][pl.ds(g * 16, 16)]
                d16 = didx[j][pl.ds(g * 16, 16)]
                e = (plsc.load_gather(as_v, [s16])
                     + plsc.load_gather(ad_v, [d16]))
                e = jnp.where(e >= 0.0, e, 0.2 * e)
                ex = jnp.exp(e)
                exb[pl.ds(g * 16, 16)] = ex
                plsc.addupdate_scatter(den_v, [d16], ex)

            def row(i16, _):
                ex16 = exb[pl.ds(i16 * 16, 16)]
                for v in range(16):
                    i = i16 * 16 + v
                    exi = ex16[v]
                    for c in range(D // 16):
                        sl = pl.ds(c * 16, 16)
                        rows[b][i, sl] = rows[b][i, sl] * exi
                return 0
            lax.fori_loop(0, G // 16, row, 0)

        def phase(s, p, w_scat, i_idx, i_gath):
            b, j = p % NB, p % NI
            pltpu.make_async_copy(h_hbm.at[sidx[j]], rows[b],
                                  semg[b]).wait()
            compute(b, j)
            pltpu.async_copy(rows[b], acc.at[didx[j]], sems[b], add=True)
            if w_scat:
                b2, j2 = (p + 2) % NB, (p + 6) % NI
                pltpu.make_async_copy(rows[b2], acc.at[didx[j2]],
                                      sems[b2]).wait()
            if i_idx:
                idx_issue(s + 3, (p + 3) % NI)
            if i_gath:
                b2, j2 = (p + 2) % NB, (p + 2) % NI
                idx_wait(j2)
                pltpu.async_copy(h_hbm.at[sidx[j2]], rows[b2], semg[b2])

        for s0 in range(3):
            idx_issue(s0, s0)
        for s0 in range(2):
            idx_wait(s0)
            pltpu.async_copy(h_hbm.at[sidx[s0]], rows[s0], semg[s0])

        for s0 in range(NI):
            phase(s0, s0, s0 >= 2, s0 + 3 < steps, s0 + 2 < steps)

        nq = steps // NI

        def superstep(q, _):
            for p in range(NI):
                phase(q * NI + p, p, True, True, True)
            return 0
        lax.fori_loop(1, nq, superstep, 0)

        for s0 in range(nq * NI, steps):
            p = s0 % NI
            phase(s0, p, True, s0 + 3 < steps, s0 + 2 < steps)
        for s0 in (steps - 2, steps - 1):
            b, j = s0 % NB, s0 % NI
            pltpu.make_async_copy(rows[b], acc.at[didx[j]],
                                  sems[b]).wait()

        pltpu.sync_copy(den_v, dparts_hbm.at[pl.ds(wid * NPAD, NPAD)])
        plsc.subcore_barrier()
        for off in range(0, RPT, G):
            rr = sid * RPT + off
            pltpu.sync_copy(acc.at[pl.ds(rr, G)], rows[0])
            pltpu.sync_copy(rows[0], num_hbm.at[cid, pl.ds(rr, G)])

    return k


def _sc_row_gather(width):
    rows = B // NW

    @functools.partial(
        pl.kernel,
        out_type=jax.ShapeDtypeStruct((B, width), jnp.float32),
        mesh=_sc_mesh(),
        scratch_types=[
            pltpu.VMEM((rows,), jnp.int32),
            pltpu.VMEM((rows, width), jnp.float32),
            pltpu.SemaphoreType.DMA,
        ],
        compiler_params=_SC_PARAMS,
    )
    def k(tab_hbm, node_hbm, out_hbm, nidx, rows_v, sem):
        cid = lax.axis_index("c")
        sid = lax.axis_index("s")
        base = (sid * NC + cid) * rows
        pltpu.sync_copy(node_hbm.at[pl.ds(base, rows)], nidx)
        pltpu.async_copy(tab_hbm.at[nidx], rows_v, sem).wait()
        pltpu.sync_copy(rows_v, out_hbm.at[pl.ds(base, rows)])

    return k


def _wext(w, a_s, a_d):
    cols = [w.T, (w.T @ a_s)[:, None], (w.T @ a_d)[:, None],
            jnp.zeros((D, D - 2), jnp.float32)]
    return jnp.concatenate(cols, axis=1)


def kernel(node_graph_feat, neighbors, node, W1, a_src1, a_dst1, b1,
           W2, a_src2, a_dst2, b2, Wc, bc):
    e_tot = neighbors.shape[1] + N
    e_pad = ((e_tot + NW * G - 1) // (NW * G)) * (NW * G)
    loop = jnp.arange(N, dtype=jnp.int32)
    src = jnp.concatenate(
        [neighbors[0], loop, jnp.zeros((e_pad - e_tot,), jnp.int32)])
    dst = jnp.concatenate(
        [neighbors[1], loop, jnp.full((e_pad - e_tot,), N, jnp.int32)])

    xp = jnp.pad(node_graph_feat, ((0, NPAD - N), (0, 0)))

    edge_pass = _edge_pass(e_pad)

    hext1 = _tc_proj(xp, _wext(W1, a_src1, a_dst1))
    num1, dp1 = edge_pass(hext1[:, :D], hext1[:ALEN, D],
                          hext1[:ALEN, D + 1], src, dst)

    hext2 = _tc_norm_mm(num1, dp1.reshape(NW, NPAD), b1,
                        _wext(W2, a_src2, a_dst2))
    num2, dp2 = edge_pass(hext2[:, :D], hext2[:ALEN, D],
                          hext2[:ALEN, D + 1], src, dst)

    wc_pad = jnp.concatenate([Wc.T, jnp.zeros((D, D - C), jnp.float32)], 1)
    bc_pad = jnp.concatenate([bc, jnp.zeros((D - C,), jnp.float32)])
    pred_all = _tc_norm_mm(num2, dp2.reshape(NW, NPAD), b2, wc_pad,
                           bias_out=bc_pad)
    pred = _sc_row_gather(D)(pred_all, node)
    return pred[:, :C]


# speedup vs baseline: 42.3646x; 1.0689x over previous
import functools

import jax
import jax.numpy as jnp
from jax import lax
from jax.experimental import pallas as pl
from jax.experimental.pallas import tpu as pltpu, tpu_sc as plsc

N = 10000
D = 128
C = 32
B = 4096

NPAD = 10240
NC = 2
NS = 16
NW = NC * NS
G = 32
NB = 4
NI = 8
RPT = NPAD // NS
DENW = 16
MRPT = NPAD // NW
ALEN = 10016


def _mm_body(x_ref, w_ref, o_ref):
    o_ref[...] = jnp.dot(x_ref[...], w_ref[...],
                         preferred_element_type=jnp.float32)


def _tc_proj(x, wext):
    mblk = NPAD // 4
    return pl.pallas_call(
        _mm_body,
        grid=(4,),
        in_specs=[
            pl.BlockSpec((mblk, D), lambda i: (i, 0)),
            pl.BlockSpec((D, 2 * D), lambda i: (0, 0)),
        ],
        out_specs=pl.BlockSpec((mblk, 2 * D), lambda i: (i, 0)),
        out_shape=jax.ShapeDtypeStruct((NPAD, 2 * D), jnp.float32),
    )(x, wext)


def _tc_norm_mm(num, dparts, b, w, bias_out=None):
    mblk = NPAD // 4
    kn = w.shape[1]

    def _x(n_ref, d_ref, b_ref):
        den = lax.dot_general(
            d_ref[...], jnp.ones((NW, 1), jnp.float32),
            (((0,), (0,)), ((), ())), preferred_element_type=jnp.float32)
        return (n_ref[0] + n_ref[1]) / (den + 1e-16) + b_ref[...]

    if bias_out is None:
        def body(n_ref, d_ref, b_ref, w_ref, o_ref):
            o_ref[...] = jnp.dot(_x(n_ref, d_ref, b_ref), w_ref[...],
                                 preferred_element_type=jnp.float32)
        extra = ()
    else:
        def body(n_ref, d_ref, b_ref, w_ref, bo_ref, o_ref):
            o_ref[...] = jnp.dot(
                _x(n_ref, d_ref, b_ref), w_ref[...],
                preferred_element_type=jnp.float32) + bo_ref[...]
        extra = (pl.BlockSpec((kn,), lambda i: (0,)),)
    args = (num, dparts, b, w) + (() if bias_out is None else (bias_out,))
    return pl.pallas_call(
        body,
        grid=(4,),
        in_specs=[
            pl.BlockSpec((NC, mblk, D), lambda i: (0, i, 0)),
            pl.BlockSpec((NW, mblk), lambda i: (0, i)),
            pl.BlockSpec((D,), lambda i: (0,)),
            pl.BlockSpec((D, kn), lambda i: (0, 0)),
        ] + list(extra),
        out_specs=pl.BlockSpec((mblk, kn), lambda i: (i, 0)),
        out_shape=jax.ShapeDtypeStruct((NPAD, kn), jnp.float32),
    )(*args)


def _sc_mesh():
    return plsc.VectorSubcoreMesh(core_axis_name="c", subcore_axis_name="s")


_SC_PARAMS = pltpu.CompilerParams(needs_layout_passes=False)


def _edge_pass(e_pad):
    chunk = e_pad // NW
    steps = chunk // G

    @functools.partial(
        pl.kernel,
        out_type=(
            jax.ShapeDtypeStruct((NC, NPAD, D), jnp.float32),
            jax.ShapeDtypeStruct((NW * NPAD,), jnp.float32),
        ),
        mesh=_sc_mesh(),
        scratch_types=[
            pltpu.VMEM((ALEN,), jnp.float32),
            pltpu.VMEM((ALEN,), jnp.float32),
            pltpu.VMEM((NPAD,), jnp.float32),
            pltpu.VMEM((G + 16,), jnp.float32),
        ]
        + [pltpu.VMEM((G, D), jnp.float32)] * NB
        + [pltpu.VMEM((G,), jnp.int32)] * NI
        + [pltpu.VMEM((G,), jnp.int32)] * NI
        + [pltpu.VMEM_SHARED((NPAD, D), jnp.float32)]
        + [pltpu.SemaphoreType.DMA] * (2 * NB + NI),
        compiler_params=_SC_PARAMS,
    )
    def k(h_hbm, as_hbm, ad_hbm, src_hbm, dst_hbm, num_hbm, dparts_hbm,
          as_v, ad_v, den_v, exb, *ring):
        rows = ring[:NB]
        sidx = ring[NB:NB + NI]
        didx = ring[NB + NI:NB + 2 * NI]
        acc = ring[NB + 2 * NI]
        semg = ring[NB + 2 * NI + 1:NB + 2 * NI + 1 + NB]
        sems = ring[NB + 2 * NI + 1 + NB:NB + 2 * NI + 1 + 2 * NB]
        semi = ring[NB + 2 * NI + 1 + 2 * NB:]
        cid = lax.axis_index("c")
        sid = lax.axis_index("s")
        wid = sid * NC + cid
        cbase = wid * chunk
        zeros16 = jnp.zeros((16,), jnp.float32)

        def zrow(i, _):
            for c in range(D // 16):
                rows[0][i, pl.ds(c * 16, 16)] = zeros16
            return 0
        lax.fori_loop(0, G, zrow, 0)
        for off in range(0, RPT, G):
            pltpu.sync_copy(rows[0], acc.at[pl.ds(sid * RPT + off, G)])

        def zden(i, _):
            den_v[pl.ds(i * 16, 16)] = zeros16
            return 0
        lax.fori_loop(0, NPAD // 16, zden, 0)
        plsc.subcore_barrier()

        pltpu.sync_copy(as_hbm, as_v)
        pltpu.sync_copy(ad_hbm, ad_v)

        def idx_issue(s, j):
            base = cbase + s * G
            pltpu.async_copy(src_hbm.at[pl.ds(base, G)], sidx[j], semi[j])
            pltpu.async_copy(dst_hbm.at[pl.ds(base, G)], didx[j], semi[j])

        def idx_wait(j):
            pltpu.make_async_copy(
                src_hbm.at[pl.ds(0, G)], sidx[j], semi[j]).wait()
            pltpu.make_async_copy(
                dst_hbm.at[pl.ds(0, G)], didx[j], semi[j]).wait()

        def compute(b, j):
            for g in range(G // 16):
                s16 = sidx[j][pl.ds(g * 16, 16)]
                d16 = didx[j][pl.ds(g * 16, 16)]
                e = (plsc.load_gather(as_v, [s16])
                     + plsc.load_gather(ad_v, [d16]))
                e = jnp.where(e >= 0.0, e, 0.2 * e)
                ex = jnp.exp(e)
                exb[pl.ds(g * 16, 16)] = ex
                plsc.addupdate_scatter(den_v, [d16], ex)

            def row(i16, _):
                ex16 = exb[pl.ds(i16 * 16, 16)]
                for v in range(16):
                    i = i16 * 16 + v
                    exi = ex16[v]
                    for c in range(D // 16):
                        sl = pl.ds(c * 16, 16)
                        rows[b][i, sl] = rows[b][i, sl] * exi
                return 0
            lax.fori_loop(0, G // 16, row, 0)

        def phase(s, p, w_scat, i_idx, i_gath):
            b, j = p % NB, p % NI
            pltpu.make_async_copy(h_hbm.at[sidx[j]], rows[b],
                                  semg[b]).wait()
            if w_scat:
                b2, j2 = (p + 2) % NB, (p + 6) % NI
                pltpu.make_async_copy(rows[b2], acc.at[didx[j2]],
                                      sems[b2]).wait()
            if i_gath:
                b2, j2 = (p + 2) % NB, (p + 2) % NI
                idx_wait(j2)
                pltpu.async_copy(h_hbm.at[sidx[j2]], rows[b2], semg[b2])
            if i_idx:
                idx_issue(s + 3, (p + 3) % NI)
            compute(b, j)
            pltpu.async_copy(rows[b], acc.at[didx[j]], sems[b], add=True)

        for s0 in range(3):
            idx_issue(s0, s0)
        for s0 in range(2):
            idx_wait(s0)
            pltpu.async_copy(h_hbm.at[sidx[s0]], rows[s0], semg[s0])

        for s0 in range(NI):
            phase(s0, s0, s0 >= 2, s0 + 3 < steps, s0 + 2 < steps)

        nq = steps // NI

        def superstep(q, _):
            for p in range(NI):
                phase(q * NI + p, p, True, True, True)
            return 0
        lax.fori_loop(1, nq, superstep, 0)

        for s0 in range(nq * NI, steps):
            p = s0 % NI
            phase(s0, p, True, s0 + 3 < steps, s0 + 2 < steps)
        for s0 in (steps - 2, steps - 1):
            b, j = s0 % NB, s0 % NI
            pltpu.make_async_copy(rows[b], acc.at[didx[j]],
                                  sems[b]).wait()

        pltpu.sync_copy(den_v, dparts_hbm.at[pl.ds(wid * NPAD, NPAD)])
        plsc.subcore_barrier()
        for off in range(0, RPT, G):
            rr = sid * RPT + off
            pltpu.sync_copy(acc.at[pl.ds(rr, G)], rows[0])
            pltpu.sync_copy(rows[0], num_hbm.at[cid, pl.ds(rr, G)])

    return k


def _sc_row_gather(width):
    rows = B // NW

    @functools.partial(
        pl.kernel,
        out_type=jax.ShapeDtypeStruct((B, width), jnp.float32),
        mesh=_sc_mesh(),
        scratch_types=[
            pltpu.VMEM((rows,), jnp.int32),
            pltpu.VMEM((rows, width), jnp.float32),
            pltpu.SemaphoreType.DMA,
        ],
        compiler_params=_SC_PARAMS,
    )
    def k(tab_hbm, node_hbm, out_hbm, nidx, rows_v, sem):
        cid = lax.axis_index("c")
        sid = lax.axis_index("s")
        base = (sid * NC + cid) * rows
        pltpu.sync_copy(node_hbm.at[pl.ds(base, rows)], nidx)
        pltpu.async_copy(tab_hbm.at[nidx], rows_v, sem).wait()
        pltpu.sync_copy(rows_v, out_hbm.at[pl.ds(base, rows)])

    return k


def _wext(w, a_s, a_d):
    cols = [w.T, (w.T @ a_s)[:, None], (w.T @ a_d)[:, None],
            jnp.zeros((D, D - 2), jnp.float32)]
    return jnp.concatenate(cols, axis=1)


def kernel(node_graph_feat, neighbors, node, W1, a_src1, a_dst1, b1,
           W2, a_src2, a_dst2, b2, Wc, bc):
    e_tot = neighbors.shape[1] + N
    e_pad = ((e_tot + NW * G - 1) // (NW * G)) * (NW * G)
    loop = jnp.arange(N, dtype=jnp.int32)
    src = jnp.concatenate(
        [neighbors[0], loop, jnp.zeros((e_pad - e_tot,), jnp.int32)])
    dst = jnp.concatenate(
        [neighbors[1], loop, jnp.full((e_pad - e_tot,), N, jnp.int32)])

    xp = jnp.pad(node_graph_feat, ((0, NPAD - N), (0, 0)))

    edge_pass = _edge_pass(e_pad)

    hext1 = _tc_proj(xp, _wext(W1, a_src1, a_dst1))
    num1, dp1 = edge_pass(hext1[:, :D], hext1[:ALEN, D],
                          hext1[:ALEN, D + 1], src, dst)

    hext2 = _tc_norm_mm(num1, dp1.reshape(NW, NPAD), b1,
                        _wext(W2, a_src2, a_dst2))
    num2, dp2 = edge_pass(hext2[:, :D], hext2[:ALEN, D],
                          hext2[:ALEN, D + 1], src, dst)

    wc_pad = jnp.concatenate([Wc.T, jnp.zeros((D, D - C), jnp.float32)], 1)
    bc_pad = jnp.concatenate([bc, jnp.zeros((D - C,), jnp.float32)])
    pred_all = _tc_norm_mm(num2, dp2.reshape(NW, NPAD), b2, wc_pad,
                           bias_out=bc_pad)
    pred = _sc_row_gather(D)(pred_all, node)
    return pred[:, :C]


# speedup vs baseline: 42.3654x; 1.0000x over previous
import functools

import jax
import jax.numpy as jnp
from jax import lax
from jax.experimental import pallas as pl
from jax.experimental.pallas import tpu as pltpu, tpu_sc as plsc

N = 10000
D = 128
C = 32
B = 4096

NPAD = 10240
NC = 2
NS = 16
NW = NC * NS
G = 32
NB = 4
NI = 8
RPT = NPAD // NS
ALEN = 10016


def _mm_body(x_ref, w_ref, o_ref):
    o_ref[...] = jnp.dot(x_ref[...], w_ref[...],
                         preferred_element_type=jnp.float32)


def _tc_proj(x, wext):
    mblk = NPAD // 4
    return pl.pallas_call(
        _mm_body,
        grid=(4,),
        in_specs=[
            pl.BlockSpec((mblk, D), lambda i: (i, 0)),
            pl.BlockSpec((D, 2 * D), lambda i: (0, 0)),
        ],
        out_specs=pl.BlockSpec((mblk, 2 * D), lambda i: (i, 0)),
        out_shape=jax.ShapeDtypeStruct((NPAD, 2 * D), jnp.float32),
    )(x, wext)


def _tc_norm_mm(num, dparts, b, w, bias_out=None):
    mblk = NPAD // 4
    kn = w.shape[1]

    def _x(n_ref, d_ref, b_ref):
        den = lax.dot_general(
            d_ref[...], jnp.ones((NW, 1), jnp.float32),
            (((0,), (0,)), ((), ())), preferred_element_type=jnp.float32)
        return (n_ref[0] + n_ref[1]) / (den + 1e-16) + b_ref[...]

    if bias_out is None:
        def body(n_ref, d_ref, b_ref, w_ref, o_ref):
            o_ref[...] = jnp.dot(_x(n_ref, d_ref, b_ref), w_ref[...],
                                 preferred_element_type=jnp.float32)
        extra = ()
    else:
        def body(n_ref, d_ref, b_ref, w_ref, bo_ref, o_ref):
            o_ref[...] = jnp.dot(
                _x(n_ref, d_ref, b_ref), w_ref[...],
                preferred_element_type=jnp.float32) + bo_ref[...]
        extra = (pl.BlockSpec((kn,), lambda i: (0,)),)
    args = (num, dparts, b, w) + (() if bias_out is None else (bias_out,))
    return pl.pallas_call(
        body,
        grid=(4,),
        in_specs=[
            pl.BlockSpec((NC, mblk, D), lambda i: (0, i, 0)),
            pl.BlockSpec((NW, mblk), lambda i: (0, i)),
            pl.BlockSpec((D,), lambda i: (0,)),
            pl.BlockSpec((D, kn), lambda i: (0, 0)),
        ] + list(extra),
        out_specs=pl.BlockSpec((mblk, kn), lambda i: (i, 0)),
        out_shape=jax.ShapeDtypeStruct((NPAD, kn), jnp.float32),
    )(*args)


def _sc_mesh():
    return plsc.VectorSubcoreMesh(core_axis_name="c", subcore_axis_name="s")


_SC_PARAMS = pltpu.CompilerParams(needs_layout_passes=False)


def _edge_pass(e_pad):
    chunk = e_pad // NW
    steps = chunk // G

    @functools.partial(
        pl.kernel,
        out_type=(
            jax.ShapeDtypeStruct((NC, NPAD, D), jnp.float32),
            jax.ShapeDtypeStruct((NW * NPAD,), jnp.float32),
        ),
        mesh=_sc_mesh(),
        scratch_types=[
            pltpu.VMEM((ALEN,), jnp.float32),
            pltpu.VMEM((ALEN,), jnp.float32),
            pltpu.VMEM((NPAD,), jnp.float32),
            pltpu.VMEM((G + 16,), jnp.float32),
        ]
        + [pltpu.VMEM((G, D), jnp.float32)] * NB
        + [pltpu.VMEM((G,), jnp.int32)] * NI
        + [pltpu.VMEM((G,), jnp.int32)] * NI
        + [pltpu.VMEM_SHARED((NPAD, D), jnp.float32)]
        + [pltpu.SemaphoreType.DMA] * (2 * NB + NI),
        compiler_params=_SC_PARAMS,
    )
    def k(h_hbm, as_hbm, ad_hbm, src_hbm, dst_hbm, num_hbm, dparts_hbm,
          as_v, ad_v, den_v, exb, *ring):
        rows = ring[:NB]
        sidx = ring[NB:NB + NI]
        didx = ring[NB + NI:NB + 2 * NI]
        acc = ring[NB + 2 * NI]
        semg = ring[NB + 2 * NI + 1:NB + 2 * NI + 1 + NB]
        sems = ring[NB + 2 * NI + 1 + NB:NB + 2 * NI + 1 + 2 * NB]
        semi = ring[NB + 2 * NI + 1 + 2 * NB:]
        cid = lax.axis_index("c")
        sid = lax.axis_index("s")
        wid = sid * NC + cid
        cbase = wid * chunk
        zeros16 = jnp.zeros((16,), jnp.float32)

        def zrow(i, _):
            for c in range(D // 16):
                rows[0][i, pl.ds(c * 16, 16)] = zeros16
            return 0
        lax.fori_loop(0, G, zrow, 0)
        for off in range(0, RPT, G):
            pltpu.sync_copy(rows[0], acc.at[pl.ds(sid * RPT + off, G)])

        def zden(i, _):
            den_v[pl.ds(i * 16, 16)] = zeros16
            return 0
        lax.fori_loop(0, NPAD // 16, zden, 0)
        plsc.subcore_barrier()

        pltpu.sync_copy(as_hbm, as_v)
        pltpu.sync_copy(ad_hbm, ad_v)

        def idx_issue(s, j):
            base = cbase + s * G
            pltpu.async_copy(src_hbm.at[pl.ds(base, G)], sidx[j], semi[j])
            pltpu.async_copy(dst_hbm.at[pl.ds(base, G)], didx[j], semi[j])

        def idx_wait(j):
            pltpu.make_async_copy(
                src_hbm.at[pl.ds(0, G)], sidx[j], semi[j]).wait()
            pltpu.make_async_copy(
                dst_hbm.at[pl.ds(0, G)], didx[j], semi[j]).wait()

        def compute(b, j):
            for g in range(G // 16):
                s16 = sidx[j][pl.ds(g * 16, 16)]
                d16 = didx[j][pl.ds(g * 16, 16)]
                e = (plsc.load_gather(as_v, [s16])
                     + plsc.load_gather(ad_v, [d16]))
                e = jnp.where(e >= 0.0, e, 0.2 * e)
                ex = jnp.exp(e)
                exb[pl.ds(g * 16, 16)] = ex
                plsc.addupdate_scatter(den_v, [d16], ex)

            def row(i16, _):
                ex16 = exb[pl.ds(i16 * 16, 16)]
                for v in range(16):
                    i = i16 * 16 + v
                    exi = ex16[v]
                    for c in range(D // 16):
                        sl = pl.ds(c * 16, 16)
                        rows[b][i, sl] = rows[b][i, sl] * exi
                return 0
            lax.fori_loop(0, G // 16, row, 0)

        def phase(s, p, w_scat, i_idx, i_gath):
            b, j = p % NB, p % NI
            pltpu.make_async_copy(h_hbm.at[sidx[j]], rows[b],
                                  semg[b]).wait()
            if w_scat:
                b2, j2 = (p + 2) % NB, (p + 6) % NI
                pltpu.make_async_copy(rows[b2], acc.at[didx[j2]],
                                      sems[b2]).wait()
            if i_gath:
                b2, j2 = (p + 2) % NB, (p + 2) % NI
                idx_wait(j2)
                pltpu.async_copy(h_hbm.at[sidx[j2]], rows[b2], semg[b2])
            if i_idx:
                idx_issue(s + 3, (p + 3) % NI)
            compute(b, j)
            pltpu.async_copy(rows[b], acc.at[didx[j]], sems[b], add=True)

        for s0 in range(3):
            idx_issue(s0, s0)
        for s0 in range(2):
            idx_wait(s0)
            pltpu.async_copy(h_hbm.at[sidx[s0]], rows[s0], semg[s0])

        for s0 in range(NI):
            phase(s0, s0, s0 >= 2, s0 + 3 < steps, s0 + 2 < steps)

        nq = steps // NI

        def superstep(q, _):
            for p in range(NI):
                phase(q * NI + p, p, True, True, True)
            return 0
        lax.fori_loop(1, nq, superstep, 0)

        for s0 in range(nq * NI, steps):
            p = s0 % NI
            phase(s0, p, True, s0 + 3 < steps, s0 + 2 < steps)
        for s0 in (steps - 2, steps - 1):
            b, j = s0 % NB, s0 % NI
            pltpu.make_async_copy(rows[b], acc.at[didx[j]],
                                  sems[b]).wait()

        pltpu.sync_copy(den_v, dparts_hbm.at[pl.ds(wid * NPAD, NPAD)])
        plsc.subcore_barrier()
        for off in range(0, RPT, G):
            rr = sid * RPT + off
            pltpu.sync_copy(acc.at[pl.ds(rr, G)], rows[0])
            pltpu.sync_copy(rows[0], num_hbm.at[cid, pl.ds(rr, G)])

    return k


def _sc_row_gather(width):
    rows = B // NW

    @functools.partial(
        pl.kernel,
        out_type=jax.ShapeDtypeStruct((B, width), jnp.float32),
        mesh=_sc_mesh(),
        scratch_types=[
            pltpu.VMEM((rows,), jnp.int32),
            pltpu.VMEM((rows, width), jnp.float32),
            pltpu.SemaphoreType.DMA,
        ],
        compiler_params=_SC_PARAMS,
    )
    def k(tab_hbm, node_hbm, out_hbm, nidx, rows_v, sem):
        cid = lax.axis_index("c")
        sid = lax.axis_index("s")
        base = (sid * NC + cid) * rows
        pltpu.sync_copy(node_hbm.at[pl.ds(base, rows)], nidx)
        pltpu.async_copy(tab_hbm.at[nidx], rows_v, sem).wait()
        pltpu.sync_copy(rows_v, out_hbm.at[pl.ds(base, rows)])

    return k


def _wext(w, a_s, a_d):
    cols = [w.T, (w.T @ a_s)[:, None], (w.T @ a_d)[:, None],
            jnp.zeros((D, D - 2), jnp.float32)]
    return jnp.concatenate(cols, axis=1)


def kernel(node_graph_feat, neighbors, node, W1, a_src1, a_dst1, b1,
           W2, a_src2, a_dst2, b2, Wc, bc):
    e_tot = neighbors.shape[1] + N
    e_pad = ((e_tot + NW * G - 1) // (NW * G)) * (NW * G)
    loop = jnp.arange(N, dtype=jnp.int32)
    src = jnp.concatenate(
        [neighbors[0], loop, jnp.zeros((e_pad - e_tot,), jnp.int32)])
    dst = jnp.concatenate(
        [neighbors[1], loop, jnp.full((e_pad - e_tot,), N, jnp.int32)])

    xp = jnp.pad(node_graph_feat, ((0, NPAD - N), (0, 0)))

    edge_pass = _edge_pass(e_pad)

    hext1 = _tc_proj(xp, _wext(W1, a_src1, a_dst1))
    num1, dp1 = edge_pass(hext1[:, :D], hext1[:ALEN, D],
                          hext1[:ALEN, D + 1], src, dst)

    hext2 = _tc_norm_mm(num1, dp1.reshape(NW, NPAD), b1,
                        _wext(W2, a_src2, a_dst2))
    num2, dp2 = edge_pass(hext2[:, :D], hext2[:ALEN, D],
                          hext2[:ALEN, D + 1], src, dst)

    wc_pad = jnp.concatenate([Wc.T, jnp.zeros((D, D - C), jnp.float32)], 1)
    bc_pad = jnp.concatenate([bc, jnp.zeros((D - C,), jnp.float32)])
    pred_all = _tc_norm_mm(num2, dp2.reshape(NW, NPAD), b2, wc_pad,
                           bias_out=bc_pad)
    pred = _sc_row_gather(D)(pred_all, node)
    return pred[:, :C]
